# baseline (speedup 1.0000x reference)
import functools

import jax
import jax.numpy as jnp
from jax import lax
from jax.experimental import pallas as pl
from jax.experimental.pallas import tpu as pltpu
from jax.experimental.pallas import tpu_sc as plsc

NC = 2
NS = 16
NW = NC * NS
L = 16
STEP = 1.0 / 65536.0

_SC_PARAMS = pltpu.CompilerParams(
    use_tc_tiling_on_sc=False,
    needs_layout_passes=False,
)


def _widx():
    return lax.axis_index("s") * NC + lax.axis_index("c")


@functools.partial(jax.jit, static_argnums=(2, 3))
def _build_maps(nei_users, nei_items, n_users, n_items):
    Bn = nei_users.shape[0]
    per = Bn // NW
    CH = 128
    nch = per // CH

    mesh = plsc.VectorSubcoreMesh(core_axis_name="c", subcore_axis_name="s")

    @functools.partial(
        pl.kernel,
        out_type=(jax.ShapeDtypeStruct((n_users,), jnp.int32),
                  jax.ShapeDtypeStruct((n_items,), jnp.int32)),
        mesh=mesh,
        compiler_params=_SC_PARAMS,
        scratch_types=[
            pltpu.VMEM((2 * nch, CH), jnp.int32),
            pltpu.VMEM((per,), jnp.int32),
            pltpu.SemaphoreType.DMA,
        ],
    )
    def build(nei_u_hbm, nei_i_hbm, map_u_hbm, map_i_hbm, idx2, vals, sem):
        base = _widx() * per
        for c in range(nch):
            pltpu.sync_copy(nei_u_hbm.at[pl.ds(base + c * CH, CH)], idx2.at[c])
            pltpu.sync_copy(nei_i_hbm.at[pl.ds(base + c * CH, CH)],
                            idx2.at[nch + c])
        for g in range(per // L):
            vals[pl.ds(g * L, L)] = base + g * L + lax.iota(jnp.int32, L)
        copies = []
        for c in range(nch):
            copies.append(pltpu.async_copy(
                vals.at[pl.ds(c * CH, CH)], map_u_hbm.at[idx2.at[c]], sem))
            copies.append(pltpu.async_copy(
                vals.at[pl.ds(c * CH, CH)], map_i_hbm.at[idx2.at[nch + c]],
                sem))
        for cp in copies:
            cp.wait()

    return build(nei_users, nei_items)


@jax.jit
def _score(u2, i2, pu2, pi2, map_u, map_i,
           nei_users, nei_items, pairs_u, pairs_i):
    P = pairs_u.shape[0]
    D = 32
    Bu = nei_users.shape[0]
    Bi = nei_items.shape[0]
    per = P // NW
    CH = 128
    nch = per // CH

    mesh = plsc.VectorSubcoreMesh(core_axis_name="c", subcore_axis_name="s")

    @functools.partial(
        pl.kernel,
        out_type=jax.ShapeDtypeStruct((P,), jnp.float32),
        mesh=mesh,
        compiler_params=_SC_PARAMS,
        scratch_types=[
            pltpu.VMEM((CH,), jnp.int32),
            pltpu.VMEM((CH,), jnp.int32),
            pltpu.VMEM((CH,), jnp.int32),
            pltpu.VMEM((CH,), jnp.int32),
            pltpu.VMEM((CH,), jnp.int32),
            pltpu.VMEM((CH,), jnp.int32),
            pltpu.VMEM((CH,), jnp.int32),
            pltpu.VMEM((CH,), jnp.int32),
            pltpu.VMEM((CH,), jnp.int32),
            pltpu.VMEM((CH,), jnp.int32),
            pltpu.VMEM((CH,), jnp.int32),
            pltpu.VMEM((CH,), jnp.int32),
            pltpu.VMEM((CH,), jnp.int32),
            pltpu.VMEM((CH,), jnp.int32),
            pltpu.VMEM((CH,), jnp.int32),
            pltpu.VMEM((CH,), jnp.int32),
            pltpu.VMEM((CH, 128), jnp.float32),
            pltpu.VMEM((CH, 128), jnp.float32),
            pltpu.VMEM((CH, 128), jnp.float32),
            pltpu.VMEM((CH, 128), jnp.float32),
            pltpu.VMEM((CH,), jnp.float32),
            pltpu.SemaphoreType.DMA,
        ],
    )
    def score(user_hbm, item_hbm, pu_hbm, pi_hbm, mu_hbm, mi_hbm,
              nu_hbm, ni_hbm, pru_hbm, pri_hbm, out_hbm,
              puv, piv, juv, jiv, guv, giv, cuv, civ, buv, biv,
              dguv, dgiv, dcuv, dciv, nuv, niv,
              urows, irows, durows, dirows, scv, sem):
        tbase = _widx() * per

        def chunk_body(c, _):
            gb = tbase + c * CH
            pltpu.sync_copy(pru_hbm.at[pl.ds(gb, CH)], puv)
            pltpu.sync_copy(pri_hbm.at[pl.ds(gb, CH)], piv)
            cp_ju = pltpu.async_copy(mu_hbm.at[puv], juv, sem)
            cp_ji = pltpu.async_copy(mi_hbm.at[piv], jiv, sem)
            for k in range(CH // L):
                sl = pl.ds(k * L, L)
                guv[sl] = lax.shift_right_logical(puv[sl], 2)
                giv[sl] = lax.shift_right_logical(piv[sl], 2)
                cuv[sl] = (puv[sl] & 3) * D
                civ[sl] = (piv[sl] & 3) * D
            cp_ur = pltpu.async_copy(user_hbm.at[guv], urows, sem)
            cp_ir = pltpu.async_copy(item_hbm.at[giv], irows, sem)
            cp_ju.wait()
            cp_ji.wait()
            for k in range(CH // L):
                sl = pl.ds(k * L, L)
                buv[sl] = jnp.minimum(jnp.maximum(juv[sl], 0), Bu - 1)
                biv[sl] = jnp.minimum(jnp.maximum(jiv[sl], 0), Bi - 1)
            cp_nu = pltpu.async_copy(nu_hbm.at[buv], nuv, sem)
            cp_ni = pltpu.async_copy(ni_hbm.at[biv], niv, sem)
            for k in range(CH // L):
                sl = pl.ds(k * L, L)
                dguv[sl] = lax.shift_right_logical(buv[sl], 2)
                dgiv[sl] = lax.shift_right_logical(biv[sl], 2)
                dcuv[sl] = (buv[sl] & 3) * D
                dciv[sl] = (biv[sl] & 3) * D
            cp_du = pltpu.async_copy(pu_hbm.at[dguv], durows, sem)
            cp_di = pltpu.async_copy(pi_hbm.at[dgiv], dirows, sem)
            cp_nu.wait()
            cp_ni.wait()
            cp_du.wait()
            cp_di.wait()
            cp_ur.wait()
            cp_ir.wait()

            def group_body(g, _):
                sl = pl.ds(g * L, L)
                msku = jnp.where(nuv[sl] == puv[sl], STEP, 0.0)
                mski = jnp.where(niv[sl] == piv[sl], STEP, 0.0)
                acc = jnp.zeros((L,), jnp.float32)
                rvec = g * L + lax.iota(jnp.int32, L)
                cu_c = cuv[sl]
                ci_c = civ[sl]
                du_c = dcuv[sl]
                di_c = dciv[sl]
                for j in range(D):
                    cu = plsc.load_gather(urows, [rvec, cu_c + j])
                    du = plsc.load_gather(durows, [rvec, du_c + j])
                    ci = plsc.load_gather(irows, [rvec, ci_c + j])
                    di = plsc.load_gather(dirows, [rvec, di_c + j])
                    acc = acc + (cu + msku * du) * (ci + mski * di)
                scv[sl] = acc
                return 0

            lax.fori_loop(0, CH // L, group_body, 0)
            pltpu.sync_copy(scv, out_hbm.at[pl.ds(gb, CH)])
            return 0

        lax.fori_loop(0, nch, chunk_body, 0)

    return score(u2, i2, pu2, pi2, map_u, map_i,
                 nei_users, nei_items, pairs_u, pairs_i)


def kernel(user_mem, item_mem, p, nei_users, nei_items, pairs_u, pairs_i):
    d = user_mem.shape[1]
    Bu = nei_users.shape[0]
    u2 = user_mem.reshape(-1, 128)
    i2 = item_mem.reshape(-1, 128)
    pu2 = p[: Bu * d].reshape(-1, 128)
    pi2 = p[Bu * d:].reshape(-1, 128)
    map_u, map_i = _build_maps(nei_users, nei_items,
                               user_mem.shape[0], item_mem.shape[0])
    return _score(u2, i2, pu2, pi2, map_u, map_i,
                  nei_users, nei_items, pairs_u, pairs_i)


# speedup vs baseline: 2.9556x; 2.9556x over previous
import functools

import jax
import jax.numpy as jnp
from jax import lax
from jax.experimental import pallas as pl
from jax.experimental.pallas import tpu as pltpu
from jax.experimental.pallas import tpu_sc as plsc
from jax._src.pallas.mosaic import sc_core

NC = 2
NS = 16
NW = NC * NS
L = 16
STEP = 1.0 / 65536.0

_SC_PARAMS = pltpu.CompilerParams(
    use_tc_tiling_on_sc=False,
    needs_layout_passes=False,
)


def _widx():
    return lax.axis_index("s") * NC + lax.axis_index("c")


def _iota():
    return lax.iota(jnp.int32, L)


@functools.partial(jax.jit, static_argnums=(2, 3))
def _build_maps(nei_users, nei_items, n_users, n_items):
    Bn = nei_users.shape[0]
    per = Bn // NW
    CH = 128
    nch = per // CH

    mesh = plsc.VectorSubcoreMesh(core_axis_name="c", subcore_axis_name="s")

    @functools.partial(
        pl.kernel,
        out_type=(jax.ShapeDtypeStruct((n_users, 2), jnp.int32),
                  jax.ShapeDtypeStruct((n_items, 2), jnp.int32)),
        mesh=mesh,
        compiler_params=_SC_PARAMS,
        scratch_types=[
            pltpu.VMEM((2 * nch, CH), jnp.int32),
            pltpu.VMEM((per, 2), jnp.int32),
            pltpu.VMEM((per, 2), jnp.int32),
            pltpu.SemaphoreType.DMA,
        ],
    )
    def build(nei_u_hbm, nei_i_hbm, map_u_hbm, map_i_hbm,
              idx2, vals_u, vals_i, sem):
        base = _widx() * per
        for c in range(nch):
            pltpu.sync_copy(nei_u_hbm.at[pl.ds(base + c * CH, CH)], idx2.at[c])
            pltpu.sync_copy(nei_i_hbm.at[pl.ds(base + c * CH, CH)],
                            idx2.at[nch + c])
        z = jnp.zeros((L,), jnp.int32)
        for g in range(per // L):
            rows = g * L + _iota()
            bvec = base + g * L + _iota()
            c, off = (g * L) // CH, (g * L) % CH
            ru = idx2[c, pl.ds(off, L)]
            ri = idx2[nch + c, pl.ds(off, L)]
            plsc.store_scatter(vals_u, [rows, z], bvec)
            plsc.store_scatter(vals_u, [rows, z + 1], ru)
            plsc.store_scatter(vals_i, [rows, z], bvec)
            plsc.store_scatter(vals_i, [rows, z + 1], ri)
        copies = []
        for c in range(nch):
            copies.append(pltpu.async_copy(
                vals_u.at[pl.ds(c * CH, CH)], map_u_hbm.at[idx2.at[c]], sem))
            copies.append(pltpu.async_copy(
                vals_i.at[pl.ds(c * CH, CH)], map_i_hbm.at[idx2.at[nch + c]],
                sem))
        for cp in copies:
            cp.wait()

    return build(nei_users, nei_items)


@jax.jit
def _score(user_mem, item_mem, p_u, p_i, map_u, map_i, pairs_u, pairs_i):
    P = pairs_u.shape[0]
    D = user_mem.shape[1]
    per = P // NW
    CH = 128
    nch = per // CH

    mesh = plsc.VectorSubcoreMesh(core_axis_name="c", subcore_axis_name="s")

    @functools.partial(
        pl.kernel,
        out_type=jax.ShapeDtypeStruct((P,), jnp.float32),
        mesh=mesh,
        compiler_params=_SC_PARAMS,
        scratch_types=[
            pltpu.VMEM((2, CH), jnp.int32),
            pltpu.VMEM((2, CH), jnp.int32),
            pltpu.VMEM((2, CH, 2), jnp.int32),
            pltpu.VMEM((2, CH, 2), jnp.int32),
            pltpu.VMEM((2, CH), jnp.int32),
            pltpu.VMEM((2, CH), jnp.int32),
            pltpu.VMEM((2, CH), jnp.float32),
            pltpu.VMEM((2, CH), jnp.float32),
            pltpu.VMEM((2, CH, 32), jnp.float32),
            pltpu.VMEM((2, CH, 32), jnp.float32),
            pltpu.VMEM((2, CH, 32), jnp.float32),
            pltpu.VMEM((2, CH, 32), jnp.float32),
            pltpu.VMEM((CH,), jnp.float32),
            pltpu.SemaphoreType.DMA((2,)),
            pltpu.SemaphoreType.DMA((2,)),
            pltpu.SemaphoreType.DMA((2,)),
        ],
    )
    def score(user_hbm, item_hbm, pu_hbm, pi_hbm, mu_hbm, mi_hbm,
              pru_hbm, pri_hbm, out_hbm,
              puv2, piv2, ju2, ji2, dbu2, dbi2, msku2, mski2,
              urows2, irows2, durows2, dirows2, scv,
              sem_map, sem_base, sem_delta):
        tbase = _widx() * per

        def front(pb, gb):
            pltpu.sync_copy(pru_hbm.at[pl.ds(gb, CH)], puv2.at[pb])
            pltpu.sync_copy(pri_hbm.at[pl.ds(gb, CH)], piv2.at[pb])
            pltpu.async_copy(mu_hbm.at[puv2.at[pb]], ju2.at[pb],
                             sem_map.at[pb])
            pltpu.async_copy(mi_hbm.at[piv2.at[pb]], ji2.at[pb],
                             sem_map.at[pb])
            pltpu.async_copy(user_hbm.at[puv2.at[pb]], urows2.at[pb],
                             sem_base.at[pb])
            pltpu.async_copy(item_hbm.at[piv2.at[pb]], irows2.at[pb],
                             sem_base.at[pb])

        def wait_map(pb):
            pltpu.make_async_copy(mu_hbm.at[puv2.at[pb]], ju2.at[pb],
                                  sem_map.at[pb]).wait()
            pltpu.make_async_copy(mi_hbm.at[piv2.at[pb]], ji2.at[pb],
                                  sem_map.at[pb]).wait()

        def wait_base(pb):
            pltpu.make_async_copy(user_hbm.at[puv2.at[pb]], urows2.at[pb],
                                  sem_base.at[pb]).wait()
            pltpu.make_async_copy(item_hbm.at[piv2.at[pb]], irows2.at[pb],
                                  sem_base.at[pb]).wait()

        def verify_and_fire_delta(pb):
            z = jnp.zeros((L,), jnp.int32)
            for g in range(CH // L):
                sl = pl.ds(g * L, L)
                rows = g * L + _iota()
                bu = plsc.load_gather(ju2.at[pb], [rows, z])
                ru = plsc.load_gather(ju2.at[pb], [rows, z + 1])
                hu = ru == puv2[pb, sl]
                dbu2[pb, sl] = jnp.where(hu, bu, -1)
                msku2[pb, sl] = jnp.where(hu, STEP, 0.0)
                bi = plsc.load_gather(ji2.at[pb], [rows, z])
                ri = plsc.load_gather(ji2.at[pb], [rows, z + 1])
                hi = ri == piv2[pb, sl]
                dbi2[pb, sl] = jnp.where(hi, bi, -1)
                mski2[pb, sl] = jnp.where(hi, STEP, 0.0)
            pltpu.async_copy(
                pu_hbm.at[sc_core.Indices(dbu2.at[pb], ignored_value=-1)],
                durows2.at[pb], sem_delta.at[pb])
            pltpu.async_copy(
                pi_hbm.at[sc_core.Indices(dbi2.at[pb], ignored_value=-1)],
                dirows2.at[pb], sem_delta.at[pb])

        def wait_delta(pb):
            pltpu.make_async_copy(
                pu_hbm.at[sc_core.Indices(dbu2.at[pb], ignored_value=-1)],
                durows2.at[pb], sem_delta.at[pb]).wait()
            pltpu.make_async_copy(
                pi_hbm.at[sc_core.Indices(dbi2.at[pb], ignored_value=-1)],
                dirows2.at[pb], sem_delta.at[pb]).wait()

        def dots(pb, gb):
            def group_body(g, _):
                sl = pl.ds(g * L, L)
                rows = g * L + _iota()
                msku = msku2[pb, sl]
                mski = mski2[pb, sl]
                acc = jnp.zeros((L,), jnp.float32)
                for j in range(D):
                    cj = jnp.full((L,), j, jnp.int32)
                    cu = plsc.load_gather(urows2.at[pb], [rows, cj])
                    du = plsc.load_gather(durows2.at[pb], [rows, cj])
                    ci = plsc.load_gather(irows2.at[pb], [rows, cj])
                    di = plsc.load_gather(dirows2.at[pb], [rows, cj])
                    acc = acc + (cu + msku * du) * (ci + mski * di)
                scv[sl] = acc
                return 0

            lax.fori_loop(0, CH // L, group_body, 0)
            pltpu.sync_copy(scv, out_hbm.at[pl.ds(gb, CH)])

        front(0, tbase)

        def step(t, _):
            ga = tbase + (2 * t) * CH
            gb = ga + CH
            gnext = jnp.minimum(gb + CH, tbase + (nch - 1) * CH)
            wait_map(0)
            verify_and_fire_delta(0)
            front(1, gb)
            wait_base(0)
            wait_delta(0)
            dots(0, ga)
            wait_map(1)
            verify_and_fire_delta(1)
            front(0, gnext)
            wait_base(1)
            wait_delta(1)
            dots(1, gb)
            return 0

        lax.fori_loop(0, nch // 2, step, 0)
        wait_map(0)
        wait_base(0)

    return score(user_mem, item_mem, p_u, p_i, map_u, map_i,
                 pairs_u, pairs_i)


def kernel(user_mem, item_mem, p, nei_users, nei_items, pairs_u, pairs_i):
    d = user_mem.shape[1]
    Bu = nei_users.shape[0]
    p_u = p[: Bu * d].reshape(Bu, d)
    p_i = p[Bu * d:].reshape(-1, d)
    map_u, map_i = _build_maps(nei_users, nei_items,
                               user_mem.shape[0], item_mem.shape[0])
    return _score(user_mem, item_mem, p_u, p_i, map_u, map_i,
                  pairs_u, pairs_i)
